# baseline (speedup 1.0000x reference)
import jax
import jax.numpy as jnp
import numpy as np
from jax import lax
from jax.experimental import pallas as pl
from jax.experimental.pallas import tpu as pltpu

B = 2
C = 96
H = 224
W = 224
E = 8
K = 2
TH = 16
NT = H // TH
HALO = TH + 6
RB = 8
NW = W // 8
EPS = float(np.finfo(float).eps)


def _rbconv(xp_ref, slab_scr, convs):
    def stage(j, carry):
        slab_scr[j] = xp_ref[0, :, pl.ds(j, W), :]
        return carry

    lax.fori_loop(0, 7, stage, 0)

    def wchunk(wc, carry):
        wof = pl.multiple_of(wc * 8, 8)

        def rowchunk(rc, carry2):
            rof = pl.multiple_of(rc * RB, RB)
            accs = [jnp.zeros((RB, 8, C), jnp.float32) for _ in convs]
            for j in range(7):
                base = slab_scr[j, pl.ds(rof, RB + 6), pl.ds(wof, 8), :]
                for ci, (_, get_wj) in enumerate(convs):
                    wv = get_wj(j)
                    for i in range(7):
                        accs[ci] = accs[ci] + base[i:i + RB] * \
                            wv[i][None, None, :]
            for (acc_ref, _), av in zip(convs, accs):
                acc_ref[pl.ds(rof, RB), pl.ds(wof, 8), :] = av
            return carry2

        return lax.fori_loop(0, TH // RB, rowchunk, carry)

    lax.fori_loop(0, NW, wchunk, 0)


def _router_body(xp_ref, rw_ref, g_ref, rb_ref, topk_ref, loss_ref,
                 logits_scr, slab_scr, acc_scr):
    b = pl.program_id(0)
    t = pl.program_id(1)

    _rbconv(xp_ref, slab_scr, [(acc_scr, lambda j: rw_ref[j])])
    acc2 = acc_scr[...].reshape(TH * W, C)
    g2 = g_ref[0]
    m2 = lax.dot_general(acc2, g2, (((0,), (0,)), ((), ())),
                         preferred_element_type=jnp.float32)
    partial = jnp.sum(m2, axis=0) + rb_ref[0, 0] * jnp.sum(g2, axis=0)

    rows = lax.broadcasted_iota(jnp.int32, (B, 16), 0)
    cols = lax.broadcasted_iota(jnp.int32, (B, 16), 1)

    @pl.when(jnp.logical_and(b == 0, t == 0))
    def _init():
        logits_scr[...] = jnp.where(cols < E, 0.0, -3.0e38)

    pad = jnp.concatenate([partial.reshape(1, E), jnp.zeros((1, 16 - E),
                                                            jnp.float32)], 1)
    logits_scr[...] += jnp.where(rows == b, pad, 0.0)

    @pl.when(jnp.logical_and(b == B - 1, t == NT - 1))
    def _gating():
        lg = logits_scr[...]
        m0 = jnp.max(lg, axis=1, keepdims=True)
        i0 = jnp.min(jnp.where(lg == m0, cols, 16), axis=1, keepdims=True)
        masked = jnp.where(cols == i0, -3.0e38, lg)
        m1 = jnp.max(masked, axis=1, keepdims=True)
        i1 = jnp.min(jnp.where(masked == m1, cols, 16), axis=1, keepdims=True)
        t1 = jnp.exp(m1 - m0)
        g0 = 1.0 / (1.0 + t1)
        g1 = t1 / (1.0 + t1)
        gates = jnp.where(cols == i0, g0, jnp.where(cols == i1, g1, 0.0))
        topk_ref[...] = jnp.where(cols == 0, i0, jnp.where(cols == 1, i1, 0)
                                  ).astype(jnp.int32)

        imp = jnp.sum(gates, axis=0)
        ld = jnp.sum((gates > 0.0).astype(jnp.float32), axis=0)

        def cv2(v):
            mean = jnp.sum(v) / E
            dv = jnp.where(cols[0] < E, v - mean, 0.0)
            return jnp.sum(dv * dv) / (E - 1) / (mean * mean + 1e-10)

        loss_ref[...] = ((cv2(imp) + cv2(ld)) * 1e-2).reshape(1, 1)


def _expert_body(xp_ref, topk_ref, we_ref, eb_ref, ws_ref, sb_ref, y_ref,
                 slab_scr, a0_scr, a1_scr, as_scr):
    b = pl.program_id(0)

    i0 = topk_ref[b, 0]
    i1 = topk_ref[b, 1]

    _rbconv(xp_ref, slab_scr,
            [(a0_scr, lambda j: we_ref[i0, j]),
             (a1_scr, lambda j: we_ref[i1, j]),
             (as_scr, lambda j: ws_ref[j])])

    st = jnp.exp(a0_scr[...] + eb_ref[i0, 0, :])
    st = st + jnp.exp(a1_scr[...] + eb_ref[i1, 0, :])
    st = jnp.where(st == 0.0, EPS, st)
    y_ref[0] = jnp.log(st) + (as_scr[...] + sb_ref[0, :])


def kernel(x, router_w, router_b, expert_w, expert_b, shared_w, shared_b,
           w_gate):
    f32 = jnp.float32
    xp = jnp.pad(jnp.transpose(x, (0, 2, 3, 1)),
                 ((0, 0), (3, 3), (3, 3), (0, 0)))
    rw = jnp.transpose(router_w.reshape(C, 7, 7), (2, 1, 0))
    g_t = w_gate.reshape(NT, TH * W, E)
    rb = router_b.reshape(1, 1)
    we = jnp.transpose(expert_w.reshape(E, C, 7, 7), (0, 3, 2, 1))
    eb = expert_b.reshape(E, 1, C)
    ws = jnp.transpose(shared_w.reshape(C, 7, 7), (2, 1, 0))
    sb = shared_b.reshape(1, C)

    topk, loss = pl.pallas_call(
        _router_body,
        grid=(B, NT),
        in_specs=[
            pl.BlockSpec((pl.Element(1), pl.Element(HALO), pl.Element(W + 6),
                          pl.Element(C)),
                         lambda b, t: (b, t * TH, 0, 0)),
            pl.BlockSpec((7, 7, C), lambda b, t: (0, 0, 0)),
            pl.BlockSpec((1, TH * W, E), lambda b, t: (t, 0, 0)),
            pl.BlockSpec(memory_space=pltpu.SMEM),
        ],
        out_specs=[
            pl.BlockSpec((B, 16), lambda b, t: (0, 0)),
            pl.BlockSpec((1, 1), lambda b, t: (0, 0)),
        ],
        out_shape=[
            jax.ShapeDtypeStruct((B, 16), jnp.int32),
            jax.ShapeDtypeStruct((1, 1), f32),
        ],
        scratch_shapes=[pltpu.VMEM((B, 16), f32),
                        pltpu.VMEM((7, HALO, W, C), f32),
                        pltpu.VMEM((TH, W, C), f32)],
    )(xp, rw, g_t, rb)

    y4 = pl.pallas_call(
        _expert_body,
        grid=(B, NT),
        in_specs=[
            pl.BlockSpec((pl.Element(1), pl.Element(HALO), pl.Element(W + 6),
                          pl.Element(C)),
                         lambda b, t: (b, t * TH, 0, 0)),
            pl.BlockSpec(memory_space=pltpu.SMEM),
            pl.BlockSpec((E, 7, 7, C), lambda b, t: (0, 0, 0, 0)),
            pl.BlockSpec((E, 1, C), lambda b, t: (0, 0, 0)),
            pl.BlockSpec((7, 7, C), lambda b, t: (0, 0, 0)),
            pl.BlockSpec((1, C), lambda b, t: (0, 0)),
        ],
        out_specs=pl.BlockSpec((1, TH, W, C), lambda b, t: (b, t, 0, 0)),
        out_shape=jax.ShapeDtypeStruct((B, H, W, C), f32),
        scratch_shapes=[pltpu.VMEM((7, HALO, W, C), f32),
                        pltpu.VMEM((TH, W, C), f32),
                        pltpu.VMEM((TH, W, C), f32),
                        pltpu.VMEM((TH, W, C), f32)],
    )(xp, topk, we, eb, ws, sb)

    y = y4.reshape(B, H * W, C)
    return y, loss.reshape(())


# speedup vs baseline: 1.2462x; 1.2462x over previous
import jax
import jax.numpy as jnp
import numpy as np
from jax import lax
from jax.experimental import pallas as pl
from jax.experimental.pallas import tpu as pltpu

B = 2
C = 96
H = 224
W = 224
E = 8
K = 2
TH = 32
NT = H // TH
HALO = TH + 6
RB = 8
NW = W // 8
EPS = float(np.finfo(float).eps)


def _rbconv(xp_ref, slab_scr, convs):
    zero = jnp.zeros((TH, W, C), jnp.float32)
    for acc_ref, _ in convs:
        acc_ref[...] = zero

    def jstep(j, carry):
        slab_scr[...] = xp_ref[0, :, pl.ds(j, W), :]
        wjs = [get_wj(j) for _, get_wj in convs]

        def wchunk(wc, carry2):
            wof = pl.multiple_of(wc * 8, 8)

            def rowchunk(rc, carry3):
                rof = pl.multiple_of(rc * RB, RB)
                base = slab_scr[pl.ds(rof, RB + 6), pl.ds(wof, 8), :]
                for (acc_ref, _), wv in zip(convs, wjs):
                    acc = acc_ref[pl.ds(rof, RB), pl.ds(wof, 8), :]
                    for i in range(7):
                        acc = acc + base[i:i + RB] * wv[i][None, None, :]
                    acc_ref[pl.ds(rof, RB), pl.ds(wof, 8), :] = acc
                return carry3

            return lax.fori_loop(0, TH // RB, rowchunk, carry2,
                                 unroll=TH // RB)

        return lax.fori_loop(0, NW, wchunk, carry, unroll=2)

    lax.fori_loop(0, 7, jstep, 0)


def _router_body(xp_ref, rw_ref, g_ref, rb_ref, topk_ref, loss_ref,
                 logits_scr, slab_scr, acc_scr):
    b = pl.program_id(0)
    t = pl.program_id(1)

    _rbconv(xp_ref, slab_scr, [(acc_scr, lambda j: rw_ref[j])])
    acc2 = acc_scr[...].reshape(TH * W, C)
    g2 = g_ref[0]
    m2 = lax.dot_general(acc2, g2, (((0,), (0,)), ((), ())),
                         preferred_element_type=jnp.float32)
    partial = jnp.sum(m2, axis=0) + rb_ref[0, 0] * jnp.sum(g2, axis=0)

    rows = lax.broadcasted_iota(jnp.int32, (B, 16), 0)
    cols = lax.broadcasted_iota(jnp.int32, (B, 16), 1)

    @pl.when(jnp.logical_and(b == 0, t == 0))
    def _init():
        logits_scr[...] = jnp.where(cols < E, 0.0, -3.0e38)

    pad = jnp.concatenate([partial.reshape(1, E), jnp.zeros((1, 16 - E),
                                                            jnp.float32)], 1)
    logits_scr[...] += jnp.where(rows == b, pad, 0.0)

    @pl.when(jnp.logical_and(b == B - 1, t == NT - 1))
    def _gating():
        lg = logits_scr[...]
        m0 = jnp.max(lg, axis=1, keepdims=True)
        i0 = jnp.min(jnp.where(lg == m0, cols, 16), axis=1, keepdims=True)
        masked = jnp.where(cols == i0, -3.0e38, lg)
        m1 = jnp.max(masked, axis=1, keepdims=True)
        i1 = jnp.min(jnp.where(masked == m1, cols, 16), axis=1, keepdims=True)
        t1 = jnp.exp(m1 - m0)
        g0 = 1.0 / (1.0 + t1)
        g1 = t1 / (1.0 + t1)
        gates = jnp.where(cols == i0, g0, jnp.where(cols == i1, g1, 0.0))
        topk_ref[...] = jnp.where(cols == 0, i0, jnp.where(cols == 1, i1, 0)
                                  ).astype(jnp.int32)

        imp = jnp.sum(gates, axis=0)
        ld = jnp.sum((gates > 0.0).astype(jnp.float32), axis=0)

        def cv2(v):
            mean = jnp.sum(v) / E
            dv = jnp.where(cols[0] < E, v - mean, 0.0)
            return jnp.sum(dv * dv) / (E - 1) / (mean * mean + 1e-10)

        loss_ref[...] = ((cv2(imp) + cv2(ld)) * 1e-2).reshape(1, 1)


def _expert_body(xp_ref, topk_ref, we_ref, eb_ref, ws_ref, sb_ref, y_ref,
                 slab_scr, a0_scr, a1_scr, as_scr):
    b = pl.program_id(0)

    i0 = topk_ref[b, 0]
    i1 = topk_ref[b, 1]

    _rbconv(xp_ref, slab_scr,
            [(a0_scr, lambda j: we_ref[i0, j]),
             (a1_scr, lambda j: we_ref[i1, j]),
             (as_scr, lambda j: ws_ref[j])])

    st = jnp.exp(a0_scr[...] + eb_ref[i0, 0, :])
    st = st + jnp.exp(a1_scr[...] + eb_ref[i1, 0, :])
    st = jnp.where(st == 0.0, EPS, st)
    y_ref[0] = jnp.log(st) + (as_scr[...] + sb_ref[0, :])


def kernel(x, router_w, router_b, expert_w, expert_b, shared_w, shared_b,
           w_gate):
    f32 = jnp.float32
    xp = jnp.pad(jnp.transpose(x, (0, 2, 3, 1)),
                 ((0, 0), (3, 3), (3, 3), (0, 0)))
    rw = jnp.transpose(router_w.reshape(C, 7, 7), (2, 1, 0))
    g_t = w_gate.reshape(NT, TH * W, E)
    rb = router_b.reshape(1, 1)
    we = jnp.transpose(expert_w.reshape(E, C, 7, 7), (0, 3, 2, 1))
    eb = expert_b.reshape(E, 1, C)
    ws = jnp.transpose(shared_w.reshape(C, 7, 7), (2, 1, 0))
    sb = shared_b.reshape(1, C)

    topk, loss = pl.pallas_call(
        _router_body,
        grid=(B, NT),
        in_specs=[
            pl.BlockSpec((pl.Element(1), pl.Element(HALO), pl.Element(W + 6),
                          pl.Element(C)),
                         lambda b, t: (b, t * TH, 0, 0)),
            pl.BlockSpec((7, 7, C), lambda b, t: (0, 0, 0)),
            pl.BlockSpec((1, TH * W, E), lambda b, t: (t, 0, 0)),
            pl.BlockSpec(memory_space=pltpu.SMEM),
        ],
        out_specs=[
            pl.BlockSpec((B, 16), lambda b, t: (0, 0)),
            pl.BlockSpec((1, 1), lambda b, t: (0, 0)),
        ],
        out_shape=[
            jax.ShapeDtypeStruct((B, 16), jnp.int32),
            jax.ShapeDtypeStruct((1, 1), f32),
        ],
        scratch_shapes=[pltpu.VMEM((B, 16), f32),
                        pltpu.VMEM((HALO, W, C), f32),
                        pltpu.VMEM((TH, W, C), f32)],
    )(xp, rw, g_t, rb)

    y4 = pl.pallas_call(
        _expert_body,
        grid=(B, NT),
        in_specs=[
            pl.BlockSpec((pl.Element(1), pl.Element(HALO), pl.Element(W + 6),
                          pl.Element(C)),
                         lambda b, t: (b, t * TH, 0, 0)),
            pl.BlockSpec(memory_space=pltpu.SMEM),
            pl.BlockSpec((E, 7, 7, C), lambda b, t: (0, 0, 0, 0)),
            pl.BlockSpec((E, 1, C), lambda b, t: (0, 0, 0)),
            pl.BlockSpec((7, 7, C), lambda b, t: (0, 0, 0)),
            pl.BlockSpec((1, C), lambda b, t: (0, 0)),
        ],
        out_specs=pl.BlockSpec((1, TH, W, C), lambda b, t: (b, t, 0, 0)),
        out_shape=jax.ShapeDtypeStruct((B, H, W, C), f32),
        scratch_shapes=[pltpu.VMEM((HALO, W, C), f32),
                        pltpu.VMEM((TH, W, C), f32),
                        pltpu.VMEM((TH, W, C), f32),
                        pltpu.VMEM((TH, W, C), f32)],
    )(xp, topk, we, eb, ws, sb)

    y = y4.reshape(B, H * W, C)
    return y, loss.reshape(())
